# baseline (speedup 1.0000x reference)
import functools

import jax
import jax.numpy as jnp
from jax import lax
from jax.experimental import pallas as pl
from jax.experimental.pallas import tpu as pltpu
from jax.experimental.pallas import tpu_sc as plsc

N_VOCAB = 364
D_MODEL = 256
B = 1024
L = 200

NC = 2
NS = 16
NW = NC * NS
ROWS = B * L
ROWS_PER_W = ROWS // NW
CHUNK = 32
N_CHUNKS = ROWS_PER_W // CHUNK

_TABLE_WORDS = N_VOCAB * D_MODEL
_CHUNK_WORDS = CHUNK * D_MODEL


def _make_emb_lookup():
  mesh = plsc.VectorSubcoreMesh(core_axis_name="c", subcore_axis_name="s")

  @functools.partial(
      pl.kernel,
      out_type=jax.ShapeDtypeStruct((ROWS * D_MODEL,), jnp.float32),
      mesh=mesh,
      scratch_types=[
          pltpu.VMEM((_TABLE_WORDS,), jnp.float32),
          pltpu.VMEM((ROWS_PER_W,), jnp.int32),
          pltpu.VMEM((_CHUNK_WORDS,), jnp.float32),
          pltpu.VMEM((_CHUNK_WORDS,), jnp.float32),
          pltpu.SemaphoreType.DMA,
          pltpu.SemaphoreType.DMA,
      ],
      compiler_params=pltpu.CompilerParams(needs_layout_passes=False),
  )
  def emb_lookup(src_hbm, emb_hbm, out_hbm, table_v, idx_v, buf_a, buf_b,
                 sem_sa, sem_sb):
    wid = lax.axis_index("s") * NC + lax.axis_index("c")
    word0 = wid * ROWS_PER_W * D_MODEL
    pltpu.sync_copy(emb_hbm, table_v)
    pltpu.sync_copy(src_hbm.at[pl.ds(wid * ROWS_PER_W, ROWS_PER_W)], idx_v)

    def assemble(k, buf):
      nj = D_MODEL // 16

      @pl.loop(0, CHUNK // 16)
      def _group(g):
        bases = idx_v[pl.ds(k * CHUNK + g * 16, 16)] * D_MODEL
        sbases = [bases[r] for r in range(16)]
        prev_vals = prev_row0 = None
        for r in range(16):
          base = sbases[r]
          row0 = g * (16 * D_MODEL) + r * D_MODEL
          vals = []
          for j in range(nj):
            vals.append(table_v[pl.ds(base + j * 16, 16)])
            if prev_vals is not None:
              buf[pl.ds(prev_row0 + j * 16, 16)] = prev_vals[j]
          prev_vals, prev_row0 = vals, row0
        for j in range(nj):
          buf[pl.ds(prev_row0 + j * 16, 16)] = prev_vals[j]

    def scatter(k, buf, sem):
      pltpu.async_copy(
          buf, out_hbm.at[pl.ds(word0 + k * _CHUNK_WORDS, _CHUNK_WORDS)], sem)

    def wait_scatter(buf, sem):
      pltpu.make_async_copy(
          buf, out_hbm.at[pl.ds(word0, _CHUNK_WORDS)], sem).wait()

    @pl.loop(0, N_CHUNKS, step=2)
    def _pair(k):
      assemble(k, buf_a)
      assemble(k + 1, buf_b)

    scatter(0, buf_a, sem_sa)
    scatter(1, buf_b, sem_sb)
    wait_scatter(buf_a, sem_sa)
    wait_scatter(buf_b, sem_sb)

  return emb_lookup


_emb_lookup = None


def kernel(src_seq, x_lengths, emb):
  del x_lengths
  global _emb_lookup
  if _emb_lookup is None:
    _emb_lookup = _make_emb_lookup()
  src_flat = src_seq.reshape(-1).astype(jnp.int32)
  emb_flat = emb.reshape(-1)
  rows = _emb_lookup(src_flat, emb_flat)
  return rows.reshape(B, L, D_MODEL).transpose(0, 2, 1)


# speedup vs baseline: 1.4958x; 1.4958x over previous
import functools

import jax
import jax.numpy as jnp
from jax import lax
from jax.experimental import pallas as pl
from jax.experimental.pallas import tpu as pltpu
from jax.experimental.pallas import tpu_sc as plsc

N_VOCAB = 364
D_MODEL = 256
B = 1024
L = 200

NC = 2
NS = 16
NW = NC * NS
ROWS = B * L
ROWS_PER_W = ROWS // NW

A_CHUNK = 32
S_CHUNK = 16
N_A_CHUNKS = 128
A_ROWS = A_CHUNK * N_A_CHUNKS
N_S_CHUNKS = (ROWS_PER_W - A_ROWS) // S_CHUNK
N_ITERS = N_S_CHUNKS
NJ = D_MODEL // 16

_TABLE_WORDS = N_VOCAB * D_MODEL
_A_WORDS = A_CHUNK * D_MODEL
_S_WORDS = S_CHUNK * D_MODEL


def _make_emb_lookup():
  mesh = plsc.VectorSubcoreMesh(core_axis_name="c", subcore_axis_name="s")

  @functools.partial(
      pl.kernel,
      out_type=jax.ShapeDtypeStruct((ROWS, D_MODEL), jnp.float32),
      mesh=mesh,
      scratch_types=[
          pltpu.VMEM((_TABLE_WORDS,), jnp.float32),
          pltpu.VMEM((ROWS_PER_W,), jnp.int32),
          pltpu.VMEM((A_CHUNK, D_MODEL), jnp.float32),
          pltpu.VMEM((A_CHUNK, D_MODEL), jnp.float32),
          pltpu.VMEM((S_CHUNK, D_MODEL), jnp.float32),
          pltpu.VMEM((S_CHUNK, D_MODEL), jnp.float32),
          pltpu.SemaphoreType.DMA,
          pltpu.SemaphoreType.DMA,
          pltpu.SemaphoreType.DMA,
          pltpu.SemaphoreType.DMA,
          pltpu.SemaphoreType.DMA,
          pltpu.SemaphoreType.DMA,
      ],
      compiler_params=pltpu.CompilerParams(needs_layout_passes=False),
  )
  def emb_lookup(src_hbm, emb2d_hbm, emb_hbm, out_hbm, table_v, idx_v,
                 buf_a, buf_b, buf_c, buf_d,
                 sem_gc, sem_gd, sem_sa, sem_sb, sem_sc, sem_sd):
    wid = lax.axis_index("s") * NC + lax.axis_index("c")
    row0 = wid * ROWS_PER_W
    pltpu.sync_copy(emb_hbm, table_v)
    pltpu.sync_copy(src_hbm.at[pl.ds(row0, ROWS_PER_W)], idx_v)

    def assemble(k, buf):
      @pl.loop(0, A_CHUNK // 16)
      def _group(g):
        bases = idx_v[pl.ds(k * A_CHUNK + g * 16, 16)] * D_MODEL
        sbases = [bases[r] for r in range(16)]
        prev_vals = prev_row0 = None
        for r in range(16):
          base = sbases[r]
          rw0 = g * 16 + r
          vals = []
          for j in range(NJ):
            vals.append(table_v[pl.ds(base + j * 16, 16)])
            if prev_vals is not None:
              buf[prev_row0, pl.ds(j * 16, 16)] = prev_vals[j]
          prev_vals, prev_row0 = vals, rw0
        for j in range(NJ):
          buf[prev_row0, pl.ds(j * 16, 16)] = prev_vals[j]

    def gather(k, buf, sem):
      srow = A_ROWS + k * S_CHUNK
      pltpu.async_copy(
          emb2d_hbm.at[idx_v.at[pl.ds(srow, S_CHUNK)]], buf, sem)

    def wait_gather(buf, sem):
      pltpu.make_async_copy(
          emb2d_hbm.at[idx_v.at[pl.ds(0, S_CHUNK)]], buf, sem).wait()

    def scatter(first_row, buf, nrows, sem):
      pltpu.async_copy(
          buf, out_hbm.at[pl.ds(row0 + first_row, nrows)], sem)

    def wait_scatter(buf, nrows, sem):
      pltpu.make_async_copy(
          buf, out_hbm.at[pl.ds(row0, nrows)], sem).wait()

    def phase(k, buf_x, buf_y, sem_gy, sem_sx, sem_sy):
      @pl.when(k > 1)
      def _():
        wait_scatter(buf_y, S_CHUNK, sem_sy)
      gather(k, buf_y, sem_gy)
      @pl.when((k > 1) & (k < N_A_CHUNKS))
      def _():
        wait_scatter(buf_x, A_CHUNK, sem_sx)
      @pl.when(k < N_A_CHUNKS)
      def _():
        assemble(k, buf_x)
        scatter(k * A_CHUNK, buf_x, A_CHUNK, sem_sx)
      wait_gather(buf_y, sem_gy)
      scatter(A_ROWS + k * S_CHUNK, buf_y, S_CHUNK, sem_sy)

    @pl.loop(0, N_ITERS, step=2)
    def _pair(k):
      phase(k, buf_a, buf_c, sem_gc, sem_sa, sem_sc)
      phase(k + 1, buf_b, buf_d, sem_gd, sem_sb, sem_sd)

    wait_scatter(buf_a, A_CHUNK, sem_sa)
    wait_scatter(buf_b, A_CHUNK, sem_sb)
    wait_scatter(buf_c, S_CHUNK, sem_sc)
    wait_scatter(buf_d, S_CHUNK, sem_sd)

  return emb_lookup


_emb_lookup = None


def kernel(src_seq, x_lengths, emb):
  del x_lengths
  global _emb_lookup
  if _emb_lookup is None:
    _emb_lookup = _make_emb_lookup()
  src_flat = src_seq.reshape(-1).astype(jnp.int32)
  emb_flat = emb.reshape(-1)
  rows = _emb_lookup(src_flat, emb, emb_flat)
  return rows.reshape(B, L, D_MODEL).transpose(0, 2, 1)


# speedup vs baseline: 1.5627x; 1.0447x over previous
import functools

import jax
import jax.numpy as jnp
from jax import lax
from jax.experimental import pallas as pl
from jax.experimental.pallas import tpu as pltpu
from jax.experimental.pallas import tpu_sc as plsc

N_VOCAB = 364
D_MODEL = 256
B = 1024
L = 200

NC = 2
NS = 16
NW = NC * NS
ROWS = B * L
ROWS_PER_W = ROWS // NW

A_CHUNK = 32
S_CHUNK = 16
N_A_CHUNKS = 133
A_ROWS = A_CHUNK * N_A_CHUNKS
N_S_CHUNKS = (ROWS_PER_W - A_ROWS) // S_CHUNK
N_ITERS = N_S_CHUNKS
NJ = D_MODEL // 16

_TABLE_WORDS = N_VOCAB * D_MODEL
_A_WORDS = A_CHUNK * D_MODEL
_S_WORDS = S_CHUNK * D_MODEL


def _make_emb_lookup():
  mesh = plsc.VectorSubcoreMesh(core_axis_name="c", subcore_axis_name="s")

  @functools.partial(
      pl.kernel,
      out_type=jax.ShapeDtypeStruct((ROWS, D_MODEL), jnp.float32),
      mesh=mesh,
      scratch_types=[
          pltpu.VMEM((_TABLE_WORDS,), jnp.float32),
          pltpu.VMEM((ROWS_PER_W,), jnp.int32),
          pltpu.VMEM((A_CHUNK, D_MODEL), jnp.float32),
          pltpu.VMEM((A_CHUNK, D_MODEL), jnp.float32),
          pltpu.VMEM((S_CHUNK, D_MODEL), jnp.float32),
          pltpu.VMEM((S_CHUNK, D_MODEL), jnp.float32),
          pltpu.SemaphoreType.DMA,
          pltpu.SemaphoreType.DMA,
          pltpu.SemaphoreType.DMA,
          pltpu.SemaphoreType.DMA,
          pltpu.SemaphoreType.DMA,
          pltpu.SemaphoreType.DMA,
      ],
      compiler_params=pltpu.CompilerParams(needs_layout_passes=False),
  )
  def emb_lookup(src_hbm, emb2d_hbm, emb_hbm, out_hbm, table_v, idx_v,
                 buf_a, buf_b, buf_c, buf_d,
                 sem_gc, sem_gd, sem_sa, sem_sb, sem_sc, sem_sd):
    wid = lax.axis_index("s") * NC + lax.axis_index("c")
    row0 = wid * ROWS_PER_W
    pltpu.sync_copy(emb_hbm, table_v)
    pltpu.sync_copy(src_hbm.at[pl.ds(row0, ROWS_PER_W)], idx_v)

    def assemble(k, buf):
      @pl.loop(0, A_CHUNK // 16)
      def _group(g):
        bases = idx_v[pl.ds(k * A_CHUNK + g * 16, 16)] * D_MODEL
        sbases = [bases[r] for r in range(16)]
        prev_vals = prev_row0 = None
        for r in range(16):
          base = sbases[r]
          rw0 = g * 16 + r
          vals = []
          for j in range(NJ):
            vals.append(table_v[pl.ds(base + j * 16, 16)])
            if prev_vals is not None:
              buf[prev_row0, pl.ds(j * 16, 16)] = prev_vals[j]
          prev_vals, prev_row0 = vals, rw0
        for j in range(NJ):
          buf[prev_row0, pl.ds(j * 16, 16)] = prev_vals[j]

    def gather(k, buf, sem):
      srow = A_ROWS + k * S_CHUNK
      pltpu.async_copy(
          emb2d_hbm.at[idx_v.at[pl.ds(srow, S_CHUNK)]], buf, sem)

    def wait_gather(buf, sem):
      pltpu.make_async_copy(
          emb2d_hbm.at[idx_v.at[pl.ds(0, S_CHUNK)]], buf, sem).wait()

    def scatter(first_row, buf, nrows, sem):
      pltpu.async_copy(
          buf, out_hbm.at[pl.ds(row0 + first_row, nrows)], sem)

    def wait_scatter(buf, nrows, sem):
      pltpu.make_async_copy(
          buf, out_hbm.at[pl.ds(row0, nrows)], sem).wait()

    def phase(k, buf_x, buf_y, sem_gy, sem_sx, sem_sy):
      @pl.when(k > 1)
      def _():
        wait_scatter(buf_y, S_CHUNK, sem_sy)
      gather(k, buf_y, sem_gy)
      @pl.when((k > 1) & (k < N_A_CHUNKS))
      def _():
        wait_scatter(buf_x, A_CHUNK, sem_sx)
      @pl.when(k < N_A_CHUNKS)
      def _():
        assemble(k, buf_x)
        scatter(k * A_CHUNK, buf_x, A_CHUNK, sem_sx)
      wait_gather(buf_y, sem_gy)
      scatter(A_ROWS + k * S_CHUNK, buf_y, S_CHUNK, sem_sy)

    @pl.loop(0, N_ITERS, step=2)
    def _pair(k):
      phase(k, buf_a, buf_c, sem_gc, sem_sa, sem_sc)
      phase(k + 1, buf_b, buf_d, sem_gd, sem_sb, sem_sd)

    wait_scatter(buf_a, A_CHUNK, sem_sa)
    wait_scatter(buf_b, A_CHUNK, sem_sb)
    wait_scatter(buf_c, S_CHUNK, sem_sc)
    wait_scatter(buf_d, S_CHUNK, sem_sd)

  return emb_lookup


_emb_lookup = None


def kernel(src_seq, x_lengths, emb):
  del x_lengths
  global _emb_lookup
  if _emb_lookup is None:
    _emb_lookup = _make_emb_lookup()
  src_flat = src_seq.reshape(-1).astype(jnp.int32)
  emb_flat = emb.reshape(-1)
  rows = _emb_lookup(src_flat, emb, emb_flat)
  return rows.reshape(B, L, D_MODEL).transpose(0, 2, 1)
